# baseline (speedup 1.0000x reference)
import functools

import jax
import jax.numpy as jnp
from jax import lax
from jax.experimental import pallas as pl
from jax.experimental.pallas import tpu as pltpu
from jax.experimental.pallas import tpu_sc as plsc

_HIDDEN = 128
_LANES = 16
_BCONST = 3.2

_MESH = plsc.VectorSubcoreMesh(
    core_axis_name="c", subcore_axis_name="s", num_cores=2, num_subcores=16
)


@functools.partial(
    pl.kernel,
    out_type=jax.ShapeDtypeStruct((1,), jnp.float32),
    mesh=_MESH,
    scratch_types=[
        pltpu.VMEM((2,), jnp.int32),
        pltpu.VMEM((2, _HIDDEN), jnp.float32),
        pltpu.VMEM((2, _HIDDEN), jnp.float32),
        pltpu.VMEM((_LANES,), jnp.float32),
        pltpu.VMEM((_LANES,), jnp.float32),
        pltpu.VMEM((_LANES,), jnp.float32),
        pltpu.SemaphoreType.DMA,
    ],
    compiler_params=pltpu.CompilerParams(needs_layout_passes=False),
)
def _autorec_sc(row, P, Q, b_u, b_i, out, idx_v, p_v, q_v, bu_v, bi_v, out_v, sem):
    wid = lax.axis_index("s") + lax.axis_index("c")

    @pl.when(wid == 0)
    def _():
        pltpu.sync_copy(row, idx_v)

        cp = pltpu.async_copy(P.at[idx_v], p_v, sem)
        cq = pltpu.async_copy(Q.at[idx_v], q_v, sem)
        cbu = pltpu.async_copy(b_u.at[idx_v], bu_v.at[pl.ds(0, 2)], sem)
        cbi = pltpu.async_copy(b_i.at[idx_v], bi_v.at[pl.ds(0, 2)], sem)
        cp.wait()
        cq.wait()

        acc = p_v[0, pl.ds(0, _LANES)] * q_v[1, pl.ds(0, _LANES)]
        for j in range(1, _HIDDEN // _LANES):
            acc = acc + p_v[0, pl.ds(j * _LANES, _LANES)] * q_v[1, pl.ds(j * _LANES, _LANES)]

        cbu.wait()
        cbi.wait()
        lane = lax.iota(jnp.int32, _LANES)
        zero = jnp.zeros((_LANES,), jnp.float32)
        acc = acc + jnp.where(lane == 0, bu_v[...], zero)
        acc = acc + jnp.where(lane == 1, bi_v[...], zero)

        total = jnp.sum(acc) + jnp.float32(_BCONST)
        out_v[...] = jnp.full((_LANES,), total, jnp.float32)
        pltpu.sync_copy(out_v.at[pl.ds(0, 1)], out)


def kernel(row, P, Q, b_u, b_i):
    return jnp.reshape(_autorec_sc(row.astype(jnp.int32), P, Q, b_u, b_i), ())


# speedup vs baseline: 1.0832x; 1.0832x over previous
import functools

import jax
import jax.numpy as jnp
from jax import lax
from jax.experimental import pallas as pl
from jax.experimental.pallas import tpu as pltpu
from jax.experimental.pallas import tpu_sc as plsc

_HIDDEN = 128
_LANES = 16
_BCONST = 3.2

_MESH = plsc.VectorSubcoreMesh(
    core_axis_name="c", subcore_axis_name="s", num_cores=1, num_subcores=1
)


@functools.partial(
    pl.kernel,
    out_type=jax.ShapeDtypeStruct((1,), jnp.float32),
    mesh=_MESH,
    scratch_types=[
        pltpu.VMEM((2,), jnp.int32),
        pltpu.VMEM((2, _HIDDEN), jnp.float32),
        pltpu.VMEM((2, _HIDDEN), jnp.float32),
        pltpu.VMEM((_LANES,), jnp.float32),
        pltpu.VMEM((_LANES,), jnp.float32),
        pltpu.VMEM((_LANES,), jnp.float32),
        pltpu.SemaphoreType.DMA,
    ],
    compiler_params=pltpu.CompilerParams(needs_layout_passes=False),
)
def _autorec_sc(row, P, Q, b_u, b_i, out, idx_v, p_v, q_v, bu_v, bi_v, out_v, sem):
    wid = lax.axis_index("s") + lax.axis_index("c")

    @pl.when(wid == 0)
    def _():
        pltpu.sync_copy(row, idx_v)

        cp = pltpu.async_copy(P.at[idx_v], p_v, sem)
        cq = pltpu.async_copy(Q.at[idx_v], q_v, sem)
        cbu = pltpu.async_copy(b_u.at[idx_v], bu_v.at[pl.ds(0, 2)], sem)
        cbi = pltpu.async_copy(b_i.at[idx_v], bi_v.at[pl.ds(0, 2)], sem)
        cp.wait()
        cq.wait()

        acc = p_v[0, pl.ds(0, _LANES)] * q_v[1, pl.ds(0, _LANES)]
        for j in range(1, _HIDDEN // _LANES):
            acc = acc + p_v[0, pl.ds(j * _LANES, _LANES)] * q_v[1, pl.ds(j * _LANES, _LANES)]

        cbu.wait()
        cbi.wait()
        lane = lax.iota(jnp.int32, _LANES)
        zero = jnp.zeros((_LANES,), jnp.float32)
        acc = acc + jnp.where(lane == 0, bu_v[...], zero)
        acc = acc + jnp.where(lane == 1, bi_v[...], zero)

        total = jnp.sum(acc) + jnp.float32(_BCONST)
        out_v[...] = jnp.full((_LANES,), total, jnp.float32)
        pltpu.sync_copy(out_v.at[pl.ds(0, 1)], out)


def kernel(row, P, Q, b_u, b_i):
    return jnp.reshape(_autorec_sc(row.astype(jnp.int32), P, Q, b_u, b_i), ())


# speedup vs baseline: 1.0924x; 1.0085x over previous
import functools

import jax
import jax.numpy as jnp
from jax import lax
from jax.experimental import pallas as pl
from jax.experimental.pallas import tpu as pltpu
from jax.experimental.pallas import tpu_sc as plsc

_HIDDEN = 128
_LANES = 16
_BCONST = 3.2

_MESH = plsc.VectorSubcoreMesh(
    core_axis_name="c", subcore_axis_name="s", num_cores=1, num_subcores=1
)


@functools.partial(
    pl.kernel,
    out_type=jax.ShapeDtypeStruct((1,), jnp.float32),
    mesh=_MESH,
    scratch_types=[
        pltpu.VMEM((2,), jnp.int32),
        pltpu.VMEM((2, _HIDDEN), jnp.float32),
        pltpu.VMEM((2, _HIDDEN), jnp.float32),
        pltpu.VMEM((_LANES,), jnp.float32),
        pltpu.VMEM((_LANES,), jnp.float32),
        pltpu.VMEM((_LANES,), jnp.float32),
        pltpu.SemaphoreType.DMA,
    ],
    compiler_params=pltpu.CompilerParams(
        needs_layout_passes=False, use_tc_tiling_on_sc=False
    ),
)
def _autorec_sc(row, P, Q, b_u, b_i, out, idx_v, p_v, q_v, bu_v, bi_v, out_v, sem):
    wid = lax.axis_index("s") + lax.axis_index("c")

    @pl.when(wid == 0)
    def _():
        pltpu.sync_copy(row, idx_v)

        cp = pltpu.async_copy(P.at[idx_v], p_v, sem)
        cq = pltpu.async_copy(Q.at[idx_v], q_v, sem)
        cbu = pltpu.async_copy(b_u.at[idx_v], bu_v.at[pl.ds(0, 2)], sem)
        cbi = pltpu.async_copy(b_i.at[idx_v], bi_v.at[pl.ds(0, 2)], sem)
        cp.wait()
        cq.wait()

        acc = p_v[0, pl.ds(0, _LANES)] * q_v[1, pl.ds(0, _LANES)]
        for j in range(1, _HIDDEN // _LANES):
            acc = acc + p_v[0, pl.ds(j * _LANES, _LANES)] * q_v[1, pl.ds(j * _LANES, _LANES)]

        cbu.wait()
        cbi.wait()
        lane = lax.iota(jnp.int32, _LANES)
        zero = jnp.zeros((_LANES,), jnp.float32)
        acc = acc + jnp.where(lane == 0, bu_v[...], zero)
        acc = acc + jnp.where(lane == 1, bi_v[...], zero)

        total = jnp.sum(acc) + jnp.float32(_BCONST)
        out_v[...] = jnp.full((_LANES,), total, jnp.float32)
        pltpu.sync_copy(out_v.at[pl.ds(0, 1)], out)


def kernel(row, P, Q, b_u, b_i):
    return jnp.reshape(_autorec_sc(row.astype(jnp.int32), P, Q, b_u, b_i), ())
